# Initial kernel scaffold; baseline (speedup 1.0000x reference)
import jax
import jax.numpy as jnp
from jax.experimental import pallas as pl


def kernel(indices, labels, table):
    raise NotImplementedError("write your pallas kernel here")


# speedup vs baseline: 2.2387x; 2.2387x over previous
import jax
import jax.numpy as jnp
from jax import lax
from jax.experimental import pallas as pl
from jax.experimental.pallas import tpu as pltpu
from jax.experimental.pallas import tpu_sc as plsc

LANES = 16
NW = 32
CHUNK = 4096


def _body(nchunk, idx_hbm, tab_hbm, coded_hbm, embed_hbm, mask_hbm,
          idx_v, coded_v, ebuf, tab_v):
    c = lax.axis_index("c")
    s = lax.axis_index("s")
    wid = s * 2 + c
    base = wid * (nchunk * CHUNK)

    pltpu.sync_copy(tab_hbm, tab_v)
    tvec = tab_v[...]

    lanes = lax.iota(jnp.int32, LANES)
    pos_off = lanes >> 2
    chan = lanes & 3

    def fill(i, carry):
        ebuf[pl.ds(i * LANES, LANES)] = tvec
        return carry
    lax.fori_loop(0, CHUNK, fill, 0)

    for ci in range(nchunk):
        pbase = base + ci * CHUNK
        pltpu.sync_copy(idx_hbm.at[pl.ds(pbase, CHUNK)], idx_v)

        def onehot(k, carry):
            gi = plsc.load_gather(idx_v, [(k << 2) + pos_off])
            vals = plsc.load_gather(tab_v, [(gi << 2) + chan])
            coded_v[pl.ds(k * LANES, LANES)] = vals
            return carry
        lax.fori_loop(0, CHUNK // 4, onehot, 0)

        pltpu.sync_copy(coded_v, coded_hbm.at[pl.ds(4 * pbase, 4 * CHUNK)])
        pltpu.sync_copy(coded_v, mask_hbm.at[pl.ds(4 * pbase, 4 * CHUNK)])
        pltpu.sync_copy(ebuf, embed_hbm.at[pl.ds(16 * pbase, 16 * CHUNK)])


def kernel(indices, labels, table):
    B, L = indices.shape
    N = B * L
    assert N % (NW * CHUNK) == 0
    nchunk = N // (NW * CHUNK)

    idx_flat = indices.reshape(N).astype(jnp.int32)
    tab_flat = table.reshape(16).astype(jnp.float32)

    mesh = plsc.VectorSubcoreMesh(core_axis_name="c", subcore_axis_name="s")
    body = lambda *args: _body(nchunk, *args)
    coded_f, embed_f, mask_f = pl.kernel(
        body,
        out_type=[
            jax.ShapeDtypeStruct((N * 4,), jnp.float32),
            jax.ShapeDtypeStruct((N * 16,), jnp.float32),
            jax.ShapeDtypeStruct((N * 4,), jnp.float32),
        ],
        mesh=mesh,
        compiler_params=pltpu.CompilerParams(needs_layout_passes=False),
        scratch_types=[
            pltpu.VMEM((CHUNK,), jnp.int32),
            pltpu.VMEM((4 * CHUNK,), jnp.float32),
            pltpu.VMEM((16 * CHUNK,), jnp.float32),
            pltpu.VMEM((LANES,), jnp.float32),
        ],
    )(idx_flat, tab_flat)

    return (coded_f.reshape(B, L, 4), embed_f.reshape(B, L, 16),
            mask_f.reshape(B, L, 4), labels)


# speedup vs baseline: 38.6935x; 17.2837x over previous
import jax
import jax.numpy as jnp
from jax import lax
from jax.experimental import pallas as pl
from jax.experimental.pallas import tpu as pltpu
from jax.experimental.pallas import tpu_sc as plsc

LANES = 16
NW = 32
BH = 256
LC = 8


def _body(B, L, idx_hbm, tab_hbm, coded_hbm, embed_hbm, mask_hbm,
          idx_blk, coded_s, ebuf, tab_v):
    c = lax.axis_index("c")
    s = lax.axis_index("s")
    wid = s * 2 + c

    pltpu.sync_copy(tab_hbm, tab_v.at[pl.ds(LANES, LANES)])

    lanes = lax.iota(jnp.int32, LANES)

    for e_hi in range(2):
        for mid in range(2 * 8):
            e = e_hi * 8 + (mid % 8)
            vec = plsc.load_gather(
                tab_v, [jnp.full((LANES,), LANES + e, jnp.int32)])
            def fill(l_i, carry, e_hi=e_hi, mid=mid, vec=vec):
                for r in range(128 // LANES):
                    ebuf[e_hi * LC + l_i, mid, pl.ds(r * LANES, LANES)] = vec
                return carry
            lax.fori_loop(0, LC, fill, 0)

    for h in range(2):
        b0 = (wid * 2 + h) * BH
        pltpu.sync_copy(idx_hbm.at[pl.ds(b0 * L, BH * L)], idx_blk)
        mid0 = (wid * 2 + h) * (BH // 128) * 4
        emid0 = (wid * 2 + h) * (BH // 128) * 8

        def chunk_body(ci, carry, mid0=mid0, emid0=emid0):
            l0 = ci * LC

            def chunk(l_i, carry2):
                l_abs = l0 + l_i
                for bt2 in range(BH // 128):
                    def group(g, carry3, bt2=bt2):
                        bidx = (bt2 * 128 + g * LANES + lanes) * L + l_abs
                        gi = plsc.load_gather(idx_blk, [bidx])
                        gi4 = gi << 2
                        for ch in range(4):
                            vals = plsc.load_gather(tab_v,
                                                    [gi4 + (LANES + ch)])
                            coded_s[l_i, bt2 * 4 + ch,
                                    pl.ds(g * LANES, LANES)] = vals
                        return carry3
                    carry2 = lax.fori_loop(0, 128 // LANES, group, carry2)
                return carry2
            lax.fori_loop(0, LC, chunk, 0)

            pltpu.sync_copy(coded_s,
                            coded_hbm.at[pl.ds(l0, LC), pl.ds(mid0, 8)])
            pltpu.sync_copy(coded_s,
                            mask_hbm.at[pl.ds(l0, LC), pl.ds(mid0, 8)])
            pltpu.sync_copy(ebuf.at[pl.ds(0, LC)],
                            embed_hbm.at[pl.ds(l0, LC), pl.ds(emid0, 16)])
            pltpu.sync_copy(ebuf.at[pl.ds(LC, LC)],
                            embed_hbm.at[pl.ds(l0, LC),
                                         pl.ds(1024 + emid0, 16)])
            return carry
        lax.fori_loop(0, L // LC, chunk_body, 0)


def kernel(indices, labels, table):
    B, L = indices.shape
    assert B % (NW * 2 * BH) == 0 or B == NW * 2 * BH
    assert L % LC == 0
    BT = B // 128

    idx_flat = indices.astype(jnp.int32).reshape(B * L)
    tab_flat = table.reshape(16).astype(jnp.float32)

    mesh = plsc.VectorSubcoreMesh(core_axis_name="c", subcore_axis_name="s")
    body = lambda *args: _body(B, L, *args)
    coded_x, embed_x, mask_x = pl.kernel(
        body,
        out_type=[
            jax.ShapeDtypeStruct((L, BT * 4, 128), jnp.float32),
            jax.ShapeDtypeStruct((L, BT * 16, 128), jnp.float32),
            jax.ShapeDtypeStruct((L, BT * 4, 128), jnp.float32),
        ],
        mesh=mesh,
        compiler_params=pltpu.CompilerParams(needs_layout_passes=False),
        scratch_types=[
            pltpu.VMEM((BH * L,), jnp.int32),
            pltpu.VMEM((LC, 8, 128), jnp.float32),
            pltpu.VMEM((2 * LC, 16, 128), jnp.float32),
            pltpu.VMEM((2 * LANES,), jnp.float32),
        ],
    )(idx_flat, tab_flat)

    coded = (coded_x.reshape(L, BT, 4, 128).transpose(1, 3, 0, 2)
             .reshape(B, L, 4))
    embed = (embed_x.reshape(L, 2, BT, 8, 128).transpose(2, 4, 0, 1, 3)
             .reshape(B, L, 16))
    mask = (mask_x.reshape(L, BT, 4, 128).transpose(1, 3, 0, 2)
            .reshape(B, L, 4))
    return coded, embed, mask, labels


# speedup vs baseline: 54.6530x; 1.4125x over previous
import jax
import jax.numpy as jnp
from jax import lax
from jax.experimental import pallas as pl
from jax.experimental.pallas import tpu as pltpu
from jax.experimental.pallas import tpu_sc as plsc

LANES = 16
NW = 32
BH = 256
LC = 10


def _body(B, L, idx_hbm, tab_hbm, coded_hbm, embed_hbm, mask_hbm,
          idx_blk, coded_s, ebuf, tab_v,
          sem_c0, sem_c1, sem_m0, sem_m1, sem_e0, sem_e1):
    c = lax.axis_index("c")
    s = lax.axis_index("s")
    wid = s * 2 + c
    sems_c = (sem_c0, sem_c1)
    sems_m = (sem_m0, sem_m1)
    sems_e = (sem_e0, sem_e1)

    pltpu.sync_copy(tab_hbm, tab_v.at[pl.ds(LANES, LANES)])

    lanes = lax.iota(jnp.int32, LANES)

    for e_hi in range(2):
        for mid in range(2 * 8):
            e = e_hi * 8 + (mid % 8)
            vec = plsc.load_gather(
                tab_v, [jnp.full((LANES,), LANES + e, jnp.int32)])
            def fill(l_i, carry, e_hi=e_hi, mid=mid, vec=vec):
                for r in range(128 // LANES):
                    ebuf[e_hi * LC + l_i, mid, pl.ds(r * LANES, LANES)] = vec
                return carry
            lax.fori_loop(0, LC, fill, 0)

    for h in range(2):
        b0 = (wid * 2 + h) * BH
        pltpu.sync_copy(idx_hbm.at[pl.ds(b0 * L, BH * L)], idx_blk)
        mid0 = (wid * 2 + h) * (BH // 128) * 4
        emid0 = (wid * 2 + h) * (BH // 128) * 8

        def pair_body(i2, carry, mid0=mid0, emid0=emid0):
            for bsel in range(2):
                ci = i2 * 2 + bsel
                l0 = ci * LC
                cbuf = coded_s.at[bsel]

                @pl.when(i2 > 0)
                def _(bsel=bsel, cbuf=cbuf):
                    pltpu.make_async_copy(
                        cbuf, coded_hbm.at[pl.ds(0, LC), pl.ds(mid0, 8)],
                        sems_c[bsel]).wait()
                    pltpu.make_async_copy(
                        cbuf, mask_hbm.at[pl.ds(0, LC), pl.ds(mid0, 8)],
                        sems_m[bsel]).wait()
                    for e_hi in range(2):
                        pltpu.make_async_copy(
                            ebuf.at[pl.ds(e_hi * LC, LC)],
                            embed_hbm.at[pl.ds(0, LC),
                                         pl.ds(e_hi * 1024 + emid0, 16)],
                            sems_e[bsel]).wait()

                def chunk(l_i, carry2, l0=l0, cbuf=cbuf):
                    l_abs = l0 + l_i
                    for bt2 in range(BH // 128):
                        def group(g, carry3, bt2=bt2, l_i=l_i,
                                  l_abs=l_abs, cbuf=cbuf):
                            bidx = (bt2 * 128 + g * LANES + lanes) * L + l_abs
                            gi = plsc.load_gather(idx_blk, [bidx])
                            gi4 = gi << 2
                            for ch in range(4):
                                vals = plsc.load_gather(
                                    tab_v, [gi4 + (LANES + ch)])
                                cbuf[l_i, bt2 * 4 + ch,
                                     pl.ds(g * LANES, LANES)] = vals
                            return carry3
                        carry2 = lax.fori_loop(0, 128 // LANES, group, carry2)
                    return carry2
                lax.fori_loop(0, LC, chunk, 0)

                pltpu.async_copy(
                    cbuf, coded_hbm.at[pl.ds(l0, LC), pl.ds(mid0, 8)],
                    sems_c[bsel])
                pltpu.async_copy(
                    cbuf, mask_hbm.at[pl.ds(l0, LC), pl.ds(mid0, 8)],
                    sems_m[bsel])
                for e_hi in range(2):
                    pltpu.async_copy(
                        ebuf.at[pl.ds(e_hi * LC, LC)],
                        embed_hbm.at[pl.ds(l0, LC),
                                     pl.ds(e_hi * 1024 + emid0, 16)],
                        sems_e[bsel])
            return carry
        lax.fori_loop(0, L // (2 * LC), pair_body, 0)

        for bsel in range(2):
            cbuf = coded_s.at[bsel]
            pltpu.make_async_copy(
                cbuf, coded_hbm.at[pl.ds(0, LC), pl.ds(mid0, 8)],
                sems_c[bsel]).wait()
            pltpu.make_async_copy(
                cbuf, mask_hbm.at[pl.ds(0, LC), pl.ds(mid0, 8)],
                sems_m[bsel]).wait()
            for e_hi in range(2):
                pltpu.make_async_copy(
                    ebuf.at[pl.ds(e_hi * LC, LC)],
                    embed_hbm.at[pl.ds(0, LC),
                                 pl.ds(e_hi * 1024 + emid0, 16)],
                    sems_e[bsel]).wait()


def kernel(indices, labels, table):
    B, L = indices.shape
    assert B == NW * 2 * BH
    assert L % (2 * LC) == 0
    BT = B // 128

    idx_flat = indices.astype(jnp.int32).reshape(B * L)
    tab_flat = table.reshape(16).astype(jnp.float32)

    mesh = plsc.VectorSubcoreMesh(core_axis_name="c", subcore_axis_name="s")
    body = lambda *args: _body(B, L, *args)
    coded_x, embed_x, mask_x = pl.kernel(
        body,
        out_type=[
            jax.ShapeDtypeStruct((L, BT * 4, 128), jnp.float32),
            jax.ShapeDtypeStruct((L, BT * 16, 128), jnp.float32),
            jax.ShapeDtypeStruct((L, BT * 4, 128), jnp.float32),
        ],
        mesh=mesh,
        compiler_params=pltpu.CompilerParams(needs_layout_passes=False),
        scratch_types=[
            pltpu.VMEM((BH * L,), jnp.int32),
            pltpu.VMEM((2, LC, 8, 128), jnp.float32),
            pltpu.VMEM((2 * LC, 16, 128), jnp.float32),
            pltpu.VMEM((2 * LANES,), jnp.float32),
            pltpu.SemaphoreType.DMA,
            pltpu.SemaphoreType.DMA,
            pltpu.SemaphoreType.DMA,
            pltpu.SemaphoreType.DMA,
            pltpu.SemaphoreType.DMA,
            pltpu.SemaphoreType.DMA,
        ],
    )(idx_flat, tab_flat)

    coded = (coded_x.reshape(L, BT, 4, 128).transpose(1, 3, 0, 2)
             .reshape(B, L, 4))
    embed = (embed_x.reshape(L, 2, BT, 8, 128).transpose(2, 4, 0, 1, 3)
             .reshape(B, L, 16))
    mask = (mask_x.reshape(L, BT, 4, 128).transpose(1, 3, 0, 2)
            .reshape(B, L, 4))
    return coded, embed, mask, labels


# speedup vs baseline: 56.3925x; 1.0318x over previous
import jax
import jax.numpy as jnp
from jax import lax
from jax.experimental import pallas as pl
from jax.experimental.pallas import tpu as pltpu
from jax.experimental.pallas import tpu_sc as plsc

LANES = 16
NW = 32
BW = 512
LC = 5


def _sc_body(B, L, idx_hbm, tab_hbm, lab_hbm, coded_hbm, mask_hbm,
             labo_hbm, idx_blk, coded_s, tab_v, lab_v,
             sem_c0, sem_c1, sem_m0, sem_m1):
    c = lax.axis_index("c")
    s = lax.axis_index("s")
    wid = s * 2 + c
    sems_c = (sem_c0, sem_c1)
    sems_m = (sem_m0, sem_m1)

    pltpu.sync_copy(lab_hbm.at[pl.ds(wid * BW, BW)], lab_v)
    pltpu.sync_copy(lab_v, labo_hbm.at[pl.ds(wid * BW, BW)])

    pltpu.sync_copy(tab_hbm, tab_v.at[pl.ds(LANES, LANES)])

    lanes = lax.iota(jnp.int32, LANES)

    b0 = wid * BW
    pltpu.sync_copy(idx_hbm.at[pl.ds(b0 * L, BW * L)], idx_blk)
    mid0 = wid * (BW // 128) * 4

    def pair_body(i2, carry):
        for bsel in range(2):
            ci = i2 * 2 + bsel
            l0 = ci * LC
            cbuf = coded_s.at[bsel]

            @pl.when(i2 > 0)
            def _(bsel=bsel, cbuf=cbuf):
                pltpu.make_async_copy(
                    cbuf, coded_hbm.at[pl.ds(0, LC), pl.ds(mid0, 16)],
                    sems_c[bsel]).wait()
                pltpu.make_async_copy(
                    cbuf, mask_hbm.at[pl.ds(0, LC), pl.ds(mid0, 16)],
                    sems_m[bsel]).wait()

            def chunk(l_i, carry2, l0=l0, cbuf=cbuf):
                l_abs = l0 + l_i
                for bt2 in range(BW // 128):
                    def group(g, carry3, bt2=bt2, l_i=l_i,
                              l_abs=l_abs, cbuf=cbuf):
                        bidx = (bt2 * 128 + g * LANES + lanes) * L + l_abs
                        gi = plsc.load_gather(idx_blk, [bidx])
                        gi4 = gi << 2
                        for ch in range(4):
                            vals = plsc.load_gather(
                                tab_v, [gi4 + (LANES + ch)])
                            cbuf[l_i, bt2 * 4 + ch,
                                 pl.ds(g * LANES, LANES)] = vals
                        return carry3
                    carry2 = lax.fori_loop(0, 128 // LANES, group, carry2)
                return carry2
            lax.fori_loop(0, LC, chunk, 0)

            pltpu.async_copy(
                cbuf, coded_hbm.at[pl.ds(l0, LC), pl.ds(mid0, 16)],
                sems_c[bsel])
            pltpu.async_copy(
                cbuf, mask_hbm.at[pl.ds(l0, LC), pl.ds(mid0, 16)],
                sems_m[bsel])
        return carry
    lax.fori_loop(0, L // (2 * LC), pair_body, 0)

    for bsel in range(2):
        cbuf = coded_s.at[bsel]
        pltpu.make_async_copy(
            cbuf, coded_hbm.at[pl.ds(0, LC), pl.ds(mid0, 16)],
            sems_c[bsel]).wait()
        pltpu.make_async_copy(
            cbuf, mask_hbm.at[pl.ds(0, LC), pl.ds(mid0, 16)],
            sems_m[bsel]).wait()


def _tc_embed_body(tab_ref, out_ref, pat_ref):
    @pl.when(pl.program_id(0) == 0)
    def _():
        mid = lax.broadcasted_iota(jnp.int32, (2048, 128), 0)
        e = (mid // 1024) * 8 + (mid % 8)
        acc = jnp.zeros((2048, 128), jnp.float32)
        for i in range(16):
            acc = jnp.where(e == i, tab_ref[i], acc)
        pat_ref[...] = acc
    out_ref[...] = pat_ref[...][None]


def kernel(indices, labels, table):
    B, L = indices.shape
    assert B == NW * BW
    assert L % (2 * LC) == 0
    BT = B // 128

    idx_flat = indices.astype(jnp.int32).reshape(B * L)
    tab_flat = table.reshape(16).astype(jnp.float32)

    mesh = plsc.VectorSubcoreMesh(core_axis_name="c", subcore_axis_name="s")
    body = lambda *args: _sc_body(B, L, *args)
    coded_x, mask_x, labels_o = pl.kernel(
        body,
        out_type=[
            jax.ShapeDtypeStruct((L, BT * 4, 128), jnp.float32),
            jax.ShapeDtypeStruct((L, BT * 4, 128), jnp.float32),
            jax.ShapeDtypeStruct((B,), jnp.float32),
        ],
        mesh=mesh,
        compiler_params=pltpu.CompilerParams(needs_layout_passes=False),
        scratch_types=[
            pltpu.VMEM((BW * L,), jnp.int32),
            pltpu.VMEM((2, LC, 16, 128), jnp.float32),
            pltpu.VMEM((2 * LANES,), jnp.float32),
            pltpu.VMEM((BW,), jnp.float32),
            pltpu.SemaphoreType.DMA,
            pltpu.SemaphoreType.DMA,
            pltpu.SemaphoreType.DMA,
            pltpu.SemaphoreType.DMA,
        ],
    )(idx_flat, tab_flat, labels)

    embed_x = pl.pallas_call(
        _tc_embed_body,
        grid=(L,),
        in_specs=[pl.BlockSpec(memory_space=pltpu.SMEM)],
        out_specs=pl.BlockSpec((1, BT * 16, 128), lambda i: (i, 0, 0)),
        out_shape=jax.ShapeDtypeStruct((L, BT * 16, 128), jnp.float32),
        scratch_shapes=[pltpu.VMEM((BT * 16, 128), jnp.float32)],
    )(tab_flat)

    coded = (coded_x.reshape(L, BT, 4, 128).transpose(1, 3, 0, 2)
             .reshape(B, L, 4))
    embed = (embed_x.reshape(L, 2, BT, 8, 128).transpose(2, 4, 0, 1, 3)
             .reshape(B, L, 16))
    mask = (mask_x.reshape(L, BT, 4, 128).transpose(1, 3, 0, 2)
            .reshape(B, L, 4))
    return coded, embed, mask, labels_o


# speedup vs baseline: 68.6545x; 1.2174x over previous
import jax
import jax.numpy as jnp
from jax import lax
from jax.experimental import pallas as pl
from jax.experimental.pallas import tpu as pltpu
from jax.experimental.pallas import tpu_sc as plsc

LANES = 16
NW = 32
BW = 512
LC = 5


def _sc_body(B, L, idx_hbm, tab_hbm, lab_hbm, coded_hbm, mask_hbm,
             labo_hbm, idx_blk, coded_s, tab_v, lab_v,
             sem_c0, sem_c1, sem_m0, sem_m1):
    c = lax.axis_index("c")
    s = lax.axis_index("s")
    wid = s * 2 + c
    sems_c = (sem_c0, sem_c1)
    sems_m = (sem_m0, sem_m1)

    pltpu.sync_copy(lab_hbm.at[pl.ds(wid * BW, BW)], lab_v)
    pltpu.sync_copy(lab_v, labo_hbm.at[pl.ds(wid * BW, BW)])

    pltpu.sync_copy(tab_hbm, tab_v.at[pl.ds(LANES, LANES)])

    lanes = lax.iota(jnp.int32, LANES)
    lanesL = lanes * L

    b0 = wid * BW
    pltpu.sync_copy(idx_hbm.at[pl.ds(b0 * L, BW * L)], idx_blk)
    mid0 = wid * (BW // 128) * 4

    def pair_body(i2, carry):
        for bsel in range(2):
            ci = i2 * 2 + bsel
            l0 = ci * LC
            cbuf = coded_s.at[bsel]

            @pl.when(i2 > 0)
            def _(bsel=bsel, cbuf=cbuf):
                pltpu.make_async_copy(
                    cbuf, coded_hbm.at[pl.ds(0, LC), pl.ds(mid0, 16)],
                    sems_c[bsel]).wait()
                pltpu.make_async_copy(
                    cbuf, mask_hbm.at[pl.ds(0, LC), pl.ds(mid0, 16)],
                    sems_m[bsel]).wait()

            def chunk(l_i, carry2, l0=l0, cbuf=cbuf):
                l_abs = l0 + l_i
                for bt2 in range(BW // 128):
                    gis = []
                    for g in range(128 // LANES):
                        bidx = (bt2 * 128 + g * LANES) * L + l_abs + lanesL
                        gis.append(plsc.load_gather(idx_blk, [bidx]) << 2)
                    for ch in range(4):
                        for g in range(128 // LANES):
                            vals = plsc.load_gather(
                                tab_v, [gis[g] + (LANES + ch)])
                            cbuf[l_i, bt2 * 4 + ch,
                                 pl.ds(g * LANES, LANES)] = vals
                return carry2
            lax.fori_loop(0, LC, chunk, 0)

            pltpu.async_copy(
                cbuf, coded_hbm.at[pl.ds(l0, LC), pl.ds(mid0, 16)],
                sems_c[bsel])
            pltpu.async_copy(
                cbuf, mask_hbm.at[pl.ds(l0, LC), pl.ds(mid0, 16)],
                sems_m[bsel])
        return carry
    lax.fori_loop(0, L // (2 * LC), pair_body, 0)

    for bsel in range(2):
        cbuf = coded_s.at[bsel]
        pltpu.make_async_copy(
            cbuf, coded_hbm.at[pl.ds(0, LC), pl.ds(mid0, 16)],
            sems_c[bsel]).wait()
        pltpu.make_async_copy(
            cbuf, mask_hbm.at[pl.ds(0, LC), pl.ds(mid0, 16)],
            sems_m[bsel]).wait()


def _tc_embed_body(tab_ref, out_ref, pat_ref):
    @pl.when(pl.program_id(0) == 0)
    def _():
        mid = lax.broadcasted_iota(jnp.int32, (2048, 128), 0)
        e = (mid // 1024) * 8 + (mid % 8)
        acc = jnp.zeros((2048, 128), jnp.float32)
        for i in range(16):
            acc = jnp.where(e == i, tab_ref[i], acc)
        pat_ref[...] = acc
    out_ref[...] = pat_ref[...][None]


def kernel(indices, labels, table):
    B, L = indices.shape
    assert B == NW * BW
    assert L % (2 * LC) == 0
    BT = B // 128

    idx_flat = indices.astype(jnp.int32).reshape(B * L)
    tab_flat = table.reshape(16).astype(jnp.float32)

    mesh = plsc.VectorSubcoreMesh(core_axis_name="c", subcore_axis_name="s")
    body = lambda *args: _sc_body(B, L, *args)
    coded_x, mask_x, labels_o = pl.kernel(
        body,
        out_type=[
            jax.ShapeDtypeStruct((L, BT * 4, 128), jnp.float32),
            jax.ShapeDtypeStruct((L, BT * 4, 128), jnp.float32),
            jax.ShapeDtypeStruct((B,), jnp.float32),
        ],
        mesh=mesh,
        compiler_params=pltpu.CompilerParams(needs_layout_passes=False),
        scratch_types=[
            pltpu.VMEM((BW * L,), jnp.int32),
            pltpu.VMEM((2, LC, 16, 128), jnp.float32),
            pltpu.VMEM((2 * LANES,), jnp.float32),
            pltpu.VMEM((BW,), jnp.float32),
            pltpu.SemaphoreType.DMA,
            pltpu.SemaphoreType.DMA,
            pltpu.SemaphoreType.DMA,
            pltpu.SemaphoreType.DMA,
        ],
    )(idx_flat, tab_flat, labels)

    embed_x = pl.pallas_call(
        _tc_embed_body,
        grid=(L,),
        in_specs=[pl.BlockSpec(memory_space=pltpu.SMEM)],
        out_specs=pl.BlockSpec((1, BT * 16, 128), lambda i: (i, 0, 0)),
        out_shape=jax.ShapeDtypeStruct((L, BT * 16, 128), jnp.float32),
        scratch_shapes=[pltpu.VMEM((BT * 16, 128), jnp.float32)],
    )(tab_flat)

    coded = (coded_x.reshape(L, BT, 4, 128).transpose(1, 3, 0, 2)
             .reshape(B, L, 4))
    embed = (embed_x.reshape(L, 2, BT, 8, 128).transpose(2, 4, 0, 1, 3)
             .reshape(B, L, 16))
    mask = (mask_x.reshape(L, BT, 4, 128).transpose(1, 3, 0, 2)
            .reshape(B, L, 4))
    return coded, embed, mask, labels_o
